# baseline (speedup 1.0000x reference)
import functools

import jax
import jax.numpy as jnp
from jax import lax
from jax.experimental import pallas as pl
from jax.experimental.pallas import tpu as pltpu
from jax.experimental.pallas import tpu_sc as plsc

S_K = 64
W_K = 128
BATCH = 16384
NC = 2
NS = 16
LANES = 16
NW = NC * NS
ROWS_PER_W = BATCH // NW
CHUNK = 128
NCHUNK = ROWS_PER_W // CHUNK

_mesh = plsc.VectorSubcoreMesh(
    core_axis_name="c", subcore_axis_name="s", num_cores=NC, num_subcores=NS
)


@functools.partial(
    pl.kernel,
    out_type=jax.ShapeDtypeStruct((BATCH,), jnp.float32),
    mesh=_mesh,
    scratch_types=[
        pltpu.VMEM((3 * CHUNK,), jnp.int32),
        pltpu.VMEM((CHUNK,), jnp.int32),
        pltpu.VMEM((CHUNK,), jnp.int32),
        pltpu.VMEM((CHUNK,), jnp.int32),
        pltpu.VMEM((CHUNK + LANES,), jnp.int32),
        pltpu.VMEM((CHUNK + LANES,), jnp.int32),
        pltpu.VMEM((CHUNK, W_K), jnp.float32),
        pltpu.VMEM((CHUNK, W_K), jnp.float32),
        pltpu.VMEM((CHUNK, W_K), jnp.float32),
        pltpu.VMEM((CHUNK,), jnp.float32),
        pltpu.SemaphoreType.DMA,
    ],
)
def _sc_forward(xflat, s_tab, r_tab, w_tab, out,
                xbuf, idx0, idx1, idx2, pb0, pb1,
                srows, rrows, wrows, outv, sem):
    wid = lax.axis_index("s") * NC + lax.axis_index("c")
    lane = lax.iota(jnp.int32, LANES)

    _dnums = lax.GatherDimensionNumbers(
        offset_dims=(), collapsed_slice_dims=(0,), start_index_map=(0,)
    )

    def _lane_shuffle(v, idx):
        return lax.gather(
            v, idx[:, None], _dnums, slice_sizes=(1,),
            mode=lax.GatherScatterMode.PROMISE_IN_BOUNDS,
        )

    for c in range(NCHUNK):
        base = wid * ROWS_PER_W + c * CHUNK
        pltpu.sync_copy(xflat.at[pl.ds(3 * base, 3 * CHUNK)], xbuf)

        def deint_body(g, carry):
            v0 = xbuf[pl.ds(3 * LANES * g, LANES)].astype(jnp.float32)
            v1 = xbuf[pl.ds(3 * LANES * g + LANES, LANES)].astype(jnp.float32)
            v2 = xbuf[pl.ds(3 * LANES * g + 2 * LANES, LANES)].astype(
                jnp.float32)
            for col, dst, pb in ((0, idx0, pb0), (1, idx1, pb1),
                                 (2, idx2, None)):
                flat = 3 * lane + col
                which = flat >> 4
                sub = flat & (LANES - 1)
                g0 = _lane_shuffle(v0, sub)
                g1 = _lane_shuffle(v1, sub)
                g2 = _lane_shuffle(v2, sub)
                res = jnp.where(which == 0, g0, jnp.where(which == 1, g1, g2))
                ri = res.astype(jnp.int32)
                if pb is None:
                    dst[pl.ds(g * LANES, LANES)] = ri
                else:
                    dst[pl.ds(g * LANES, LANES)] = ri >> 1
                    pb[pl.ds(g * LANES, LANES)] = (ri & 1) << 6
            return carry

        lax.fori_loop(0, CHUNK // LANES, deint_body, 0)

        cs = pltpu.async_copy(s_tab.at[idx0], srows, sem)
        cr = pltpu.async_copy(r_tab.at[idx1], rrows, sem)
        cw = pltpu.async_copy(w_tab.at[idx2], wrows, sem)
        cs.wait()
        cr.wait()
        cw.wait()

        def group_body(g, carry):
            def row_body(j, acc_out):
                i = g * LANES + j
                cs_ = pb0[pl.ds(i, LANES)][0]
                cr_ = pb1[pl.ds(i, LANES)][0]
                acc = (srows[i, pl.ds(cs_, LANES)]
                       * wrows[i, pl.ds(0, LANES)])
                for k in range(1, S_K // LANES):
                    acc = acc + (srows[i, pl.ds(cs_ + k * LANES, LANES)]
                                 * wrows[i, pl.ds(k * LANES, LANES)])
                for k in range(S_K // LANES):
                    acc = acc + (rrows[i, pl.ds(cr_ + k * LANES, LANES)]
                                 * wrows[i, pl.ds(S_K + k * LANES, LANES)])
                for d in (8, 4, 2, 1):
                    acc = acc + _lane_shuffle(acc, lane ^ d)
                return jnp.where(lane == j, acc, acc_out)

            accs = lax.fori_loop(
                0, LANES, row_body, jnp.zeros((LANES,), jnp.float32)
            )
            outv[pl.ds(g * LANES, LANES)] = 1.0 / (1.0 + jnp.exp(-accs))
            return carry

        lax.fori_loop(0, CHUNK // LANES, group_body, 0)
        pltpu.sync_copy(outv, out.at[pl.ds(base, CHUNK)])


def kernel(X, s_embeds, r_embeds, w_embeds):
    xflat = X.astype(jnp.int32).reshape(-1)
    s_tab = s_embeds.reshape(s_embeds.shape[0] // 2, 2 * s_embeds.shape[1])
    r_tab = r_embeds.reshape(r_embeds.shape[0] // 2, 2 * r_embeds.shape[1])
    return _sc_forward(xflat, s_tab, r_tab, w_embeds)


# speedup vs baseline: 1.1717x; 1.1717x over previous
import functools

import jax
import jax.numpy as jnp
from jax import lax
from jax.experimental import pallas as pl
from jax.experimental.pallas import tpu as pltpu
from jax.experimental.pallas import tpu_sc as plsc

S_K = 64
W_K = 128
BATCH = 16384
NC = 2
NS = 16
LANES = 16
NW = NC * NS
ROWS_PER_W = BATCH // NW
CHUNK = 128
NCHUNK = ROWS_PER_W // CHUNK

_mesh = plsc.VectorSubcoreMesh(
    core_axis_name="c", subcore_axis_name="s", num_cores=NC, num_subcores=NS
)


@functools.partial(
    pl.kernel,
    out_type=jax.ShapeDtypeStruct((BATCH,), jnp.float32),
    mesh=_mesh,
    scratch_types=[
        pltpu.VMEM((3 * CHUNK,), jnp.int32),
        pltpu.VMEM((CHUNK,), jnp.int32),
        pltpu.VMEM((CHUNK,), jnp.int32),
        pltpu.VMEM((CHUNK,), jnp.int32),
        pltpu.VMEM((CHUNK, W_K), jnp.float32),
        pltpu.VMEM((CHUNK, W_K), jnp.float32),
        pltpu.VMEM((CHUNK, W_K), jnp.float32),
        pltpu.VMEM((CHUNK,), jnp.float32),
        pltpu.SemaphoreType.DMA,
    ],
)
def _sc_forward(xflat, sr_tab, w_tab, out,
                xbuf, idx0, idx1, idx2, srows, rrows, wrows, outv, sem):
    wid = lax.axis_index("s") * NC + lax.axis_index("c")
    lane = lax.iota(jnp.int32, LANES)

    _dnums = lax.GatherDimensionNumbers(
        offset_dims=(), collapsed_slice_dims=(0,), start_index_map=(0,)
    )

    def _lane_shuffle(v, idx):
        return lax.gather(
            v, idx[:, None], _dnums, slice_sizes=(1,),
            mode=lax.GatherScatterMode.PROMISE_IN_BOUNDS,
        )

    for c in range(NCHUNK):
        base = wid * ROWS_PER_W + c * CHUNK
        pltpu.sync_copy(xflat.at[pl.ds(3 * base, 3 * CHUNK)], xbuf)

        def deint_body(g, carry):
            v0 = xbuf[pl.ds(3 * LANES * g, LANES)].astype(jnp.float32)
            v1 = xbuf[pl.ds(3 * LANES * g + LANES, LANES)].astype(jnp.float32)
            v2 = xbuf[pl.ds(3 * LANES * g + 2 * LANES, LANES)].astype(
                jnp.float32)
            for col, dst in ((0, idx0), (1, idx1), (2, idx2)):
                flat = 3 * lane + col
                which = flat >> 4
                sub = flat & (LANES - 1)
                g0 = _lane_shuffle(v0, sub)
                g1 = _lane_shuffle(v1, sub)
                g2 = _lane_shuffle(v2, sub)
                res = jnp.where(which == 0, g0, jnp.where(which == 1, g1, g2))
                dst[pl.ds(g * LANES, LANES)] = res.astype(jnp.int32)
            return carry

        lax.fori_loop(0, CHUNK // LANES, deint_body, 0)

        cs = pltpu.async_copy(sr_tab.at[idx0], srows, sem)
        cr = pltpu.async_copy(sr_tab.at[idx1], rrows, sem)
        cw = pltpu.async_copy(w_tab.at[idx2], wrows, sem)
        cs.wait()
        cr.wait()
        cw.wait()

        def group_body(g, carry):
            def row_body(j, acc_out):
                i = g * LANES + j
                acc = srows[i, pl.ds(0, LANES)] * wrows[i, pl.ds(0, LANES)]
                for k in range(1, S_K // LANES):
                    acc = acc + (srows[i, pl.ds(k * LANES, LANES)]
                                 * wrows[i, pl.ds(k * LANES, LANES)])
                for k in range(S_K // LANES):
                    acc = acc + (rrows[i, pl.ds(S_K + k * LANES, LANES)]
                                 * wrows[i, pl.ds(S_K + k * LANES, LANES)])
                for d in (8, 4, 2, 1):
                    acc = acc + _lane_shuffle(acc, lane ^ d)
                return jnp.where(lane == j, acc, acc_out)

            accs = lax.fori_loop(
                0, LANES, row_body, jnp.zeros((LANES,), jnp.float32)
            )
            outv[pl.ds(g * LANES, LANES)] = 1.0 / (1.0 + jnp.exp(-accs))
            return carry

        lax.fori_loop(0, CHUNK // LANES, group_body, 0)
        pltpu.sync_copy(outv, out.at[pl.ds(base, CHUNK)])


def kernel(X, s_embeds, r_embeds, w_embeds):
    xflat = X.astype(jnp.int32).reshape(-1)
    sr_tab = jnp.concatenate([s_embeds, r_embeds], axis=1)
    return _sc_forward(xflat, sr_tab, w_embeds)


# speedup vs baseline: 1.2989x; 1.1086x over previous
import functools

import jax
import jax.numpy as jnp
from jax import lax
from jax.experimental import pallas as pl
from jax.experimental.pallas import tpu as pltpu
from jax.experimental.pallas import tpu_sc as plsc

S_K = 64
W_K = 128
BATCH = 16384
NC = 2
NS = 16
LANES = 16
NW = NC * NS
ROWS_PER_W = BATCH // NW
CHUNK = 128
NCHUNK = ROWS_PER_W // CHUNK

_mesh = plsc.VectorSubcoreMesh(
    core_axis_name="c", subcore_axis_name="s", num_cores=NC, num_subcores=NS
)


@functools.partial(
    pl.kernel,
    out_type=jax.ShapeDtypeStruct((BATCH,), jnp.float32),
    mesh=_mesh,
    scratch_types=[
        pltpu.VMEM((3 * CHUNK + LANES,), jnp.int32),
        pltpu.VMEM((CHUNK,), jnp.int32),
        pltpu.VMEM((CHUNK, S_K), jnp.float32),
        pltpu.VMEM((CHUNK, S_K), jnp.float32),
        pltpu.VMEM((CHUNK, W_K), jnp.float32),
        pltpu.VMEM((CHUNK,), jnp.float32),
        pltpu.SemaphoreType.DMA,
        pltpu.SemaphoreType.DMA,
        pltpu.SemaphoreType.DMA,
    ],
)
def _sc_forward(xflat, s_tab, r_tab, w_tab, out,
                xbuf, idx2, srows, rrows, wrows, outv,
                sem_s, sem_r, sem_w):
    wid = lax.axis_index("s") * NC + lax.axis_index("c")
    lane = lax.iota(jnp.int32, LANES)

    _dnums = lax.GatherDimensionNumbers(
        offset_dims=(), collapsed_slice_dims=(0,), start_index_map=(0,)
    )

    def _lane_shuffle(v, idx):
        return lax.gather(
            v, idx[:, None], _dnums, slice_sizes=(1,),
            mode=lax.GatherScatterMode.PROMISE_IN_BOUNDS,
        )

    for c in range(NCHUNK):
        base = wid * ROWS_PER_W + c * CHUNK
        pltpu.sync_copy(xflat.at[pl.ds(3 * base, 3 * CHUNK)],
                        xbuf.at[pl.ds(0, 3 * CHUNK)])

        def dma_body(j, carry):
            i0 = xbuf[pl.ds(3 * j, LANES)][0]
            i1 = xbuf[pl.ds(3 * j + 1, LANES)][0]
            pltpu.async_copy(s_tab.at[pl.ds(i0, 1), :],
                             srows.at[pl.ds(j, 1), :], sem_s)
            pltpu.async_copy(r_tab.at[pl.ds(i1, 1), :],
                             rrows.at[pl.ds(j, 1), :], sem_r)
            return carry

        lax.fori_loop(0, CHUNK, dma_body, 0)

        def deint_body(g, carry):
            v0 = xbuf[pl.ds(3 * LANES * g, LANES)].astype(jnp.float32)
            v1 = xbuf[pl.ds(3 * LANES * g + LANES, LANES)].astype(jnp.float32)
            v2 = xbuf[pl.ds(3 * LANES * g + 2 * LANES, LANES)].astype(
                jnp.float32)
            flat = 3 * lane + 2
            which = flat >> 4
            sub = flat & (LANES - 1)
            g0 = _lane_shuffle(v0, sub)
            g1 = _lane_shuffle(v1, sub)
            g2 = _lane_shuffle(v2, sub)
            res = jnp.where(which == 0, g0, jnp.where(which == 1, g1, g2))
            idx2[pl.ds(g * LANES, LANES)] = res.astype(jnp.int32)
            return carry

        lax.fori_loop(0, CHUNK // LANES, deint_body, 0)
        cw = pltpu.async_copy(w_tab.at[idx2], wrows, sem_w)

        pltpu.make_async_copy(s_tab.at[pl.ds(0, CHUNK), :], srows,
                              sem_s).wait()
        pltpu.make_async_copy(r_tab.at[pl.ds(0, CHUNK), :], rrows,
                              sem_r).wait()
        cw.wait()

        def group_body(g, carry):
            def row_body(j, acc_out):
                i = g * LANES + j
                acc = srows[i, pl.ds(0, LANES)] * wrows[i, pl.ds(0, LANES)]
                for k in range(1, S_K // LANES):
                    acc = acc + (srows[i, pl.ds(k * LANES, LANES)]
                                 * wrows[i, pl.ds(k * LANES, LANES)])
                for k in range(S_K // LANES):
                    acc = acc + (rrows[i, pl.ds(k * LANES, LANES)]
                                 * wrows[i, pl.ds(S_K + k * LANES, LANES)])
                for d in (8, 4, 2, 1):
                    acc = acc + _lane_shuffle(acc, lane ^ d)
                return jnp.where(lane == j, acc, acc_out)

            accs = lax.fori_loop(
                0, LANES, row_body, jnp.zeros((LANES,), jnp.float32)
            )
            outv[pl.ds(g * LANES, LANES)] = 1.0 / (1.0 + jnp.exp(-accs))
            return carry

        lax.fori_loop(0, CHUNK // LANES, group_body, 0)
        pltpu.sync_copy(outv, out.at[pl.ds(base, CHUNK)])


def kernel(X, s_embeds, r_embeds, w_embeds):
    xflat = X.astype(jnp.int32).reshape(-1)
    return _sc_forward(xflat, s_embeds, r_embeds, w_embeds)


# speedup vs baseline: 1.3575x; 1.0451x over previous
import functools

import jax
import jax.numpy as jnp
from jax import lax
from jax.experimental import pallas as pl
from jax.experimental.pallas import tpu as pltpu
from jax.experimental.pallas import tpu_sc as plsc

S_K = 64
W_K = 128
BATCH = 16384
NC = 2
NS = 16
LANES = 16
NW = NC * NS
ROWS_PER_W = BATCH // NW
CHUNK = 128
NCHUNK = ROWS_PER_W // CHUNK
NSLOT = 2

_mesh = plsc.VectorSubcoreMesh(
    core_axis_name="c", subcore_axis_name="s", num_cores=NC, num_subcores=NS
)

_slot_scratch = [
    pltpu.VMEM((3 * CHUNK + LANES,), jnp.int32),
    pltpu.VMEM((CHUNK,), jnp.int32),
    pltpu.VMEM((CHUNK, S_K), jnp.float32),
    pltpu.VMEM((CHUNK, S_K), jnp.float32),
    pltpu.VMEM((CHUNK, W_K), jnp.float32),
    pltpu.SemaphoreType.DMA,
    pltpu.SemaphoreType.DMA,
    pltpu.SemaphoreType.DMA,
]


@functools.partial(
    pl.kernel,
    out_type=jax.ShapeDtypeStruct((BATCH,), jnp.float32),
    mesh=_mesh,
    scratch_types=_slot_scratch * NSLOT + [
        pltpu.VMEM((CHUNK,), jnp.float32),
    ],
)
def _sc_forward(xflat, s_tab, r_tab, w_tab, out, *scratch):
    nper = len(_slot_scratch)
    slots = [scratch[i * nper:(i + 1) * nper] for i in range(NSLOT)]
    outv = scratch[NSLOT * nper]

    wid = lax.axis_index("s") * NC + lax.axis_index("c")
    lane = lax.iota(jnp.int32, LANES)

    _dnums = lax.GatherDimensionNumbers(
        offset_dims=(), collapsed_slice_dims=(0,), start_index_map=(0,)
    )

    def _lane_shuffle(v, idx):
        return lax.gather(
            v, idx[:, None], _dnums, slice_sizes=(1,),
            mode=lax.GatherScatterMode.PROMISE_IN_BOUNDS,
        )

    def issue(c, slot):
        xbuf, idx2, srows, rrows, wrows, sem_s, sem_r, sem_w = slot
        base = wid * ROWS_PER_W + c * CHUNK
        pltpu.sync_copy(xflat.at[pl.ds(3 * base, 3 * CHUNK)],
                        xbuf.at[pl.ds(0, 3 * CHUNK)])

        def deint_body(g, carry):
            v0 = xbuf[pl.ds(3 * LANES * g, LANES)].astype(jnp.float32)
            v1 = xbuf[pl.ds(3 * LANES * g + LANES, LANES)].astype(jnp.float32)
            v2 = xbuf[pl.ds(3 * LANES * g + 2 * LANES, LANES)].astype(
                jnp.float32)
            flat = 3 * lane + 2
            which = flat >> 4
            sub = flat & (LANES - 1)
            g0 = _lane_shuffle(v0, sub)
            g1 = _lane_shuffle(v1, sub)
            g2 = _lane_shuffle(v2, sub)
            res = jnp.where(which == 0, g0, jnp.where(which == 1, g1, g2))
            idx2[pl.ds(g * LANES, LANES)] = res.astype(jnp.int32)
            return carry

        lax.fori_loop(0, CHUNK // LANES, deint_body, 0)
        pltpu.async_copy(w_tab.at[idx2], wrows, sem_w)

        def dma_body(j, carry):
            i0 = xbuf[pl.ds(3 * j, LANES)][0]
            i1 = xbuf[pl.ds(3 * j + 1, LANES)][0]
            pltpu.async_copy(s_tab.at[pl.ds(i0, 1), :],
                             srows.at[pl.ds(j, 1), :], sem_s)
            pltpu.async_copy(r_tab.at[pl.ds(i1, 1), :],
                             rrows.at[pl.ds(j, 1), :], sem_r)
            return carry

        lax.fori_loop(0, CHUNK, dma_body, 0)

    def drain(slot):
        _, _, srows, rrows, wrows, sem_s, sem_r, sem_w = slot
        pltpu.make_async_copy(s_tab.at[pl.ds(0, CHUNK), :], srows,
                              sem_s).wait()
        pltpu.make_async_copy(r_tab.at[pl.ds(0, CHUNK), :], rrows,
                              sem_r).wait()
        pltpu.make_async_copy(w_tab.at[pl.ds(0, CHUNK), :], wrows,
                              sem_w).wait()

    def compute(c, slot):
        _, _, srows, rrows, wrows, _, _, _ = slot
        base = wid * ROWS_PER_W + c * CHUNK

        def group_body(g, carry):
            def row_body(j, acc_out):
                i = g * LANES + j
                acc = srows[i, pl.ds(0, LANES)] * wrows[i, pl.ds(0, LANES)]
                for k in range(1, S_K // LANES):
                    acc = acc + (srows[i, pl.ds(k * LANES, LANES)]
                                 * wrows[i, pl.ds(k * LANES, LANES)])
                for k in range(S_K // LANES):
                    acc = acc + (rrows[i, pl.ds(k * LANES, LANES)]
                                 * wrows[i, pl.ds(S_K + k * LANES, LANES)])
                for d in (8, 4, 2, 1):
                    acc = acc + _lane_shuffle(acc, lane ^ d)
                return jnp.where(lane == j, acc, acc_out)

            accs = lax.fori_loop(
                0, LANES, row_body, jnp.zeros((LANES,), jnp.float32)
            )
            outv[pl.ds(g * LANES, LANES)] = 1.0 / (1.0 + jnp.exp(-accs))
            return carry

        lax.fori_loop(0, CHUNK // LANES, group_body, 0)
        pltpu.sync_copy(outv, out.at[pl.ds(base, CHUNK)])

    issue(0, slots[0])
    for c in range(NCHUNK):
        if c + 1 < NCHUNK:
            issue(c + 1, slots[(c + 1) % NSLOT])
        drain(slots[c % NSLOT])
        compute(c, slots[c % NSLOT])


def kernel(X, s_embeds, r_embeds, w_embeds):
    xflat = X.astype(jnp.int32).reshape(-1)
    return _sc_forward(xflat, s_embeds, r_embeds, w_embeds)


# speedup vs baseline: 1.5158x; 1.1167x over previous
import functools

import jax
import jax.numpy as jnp
from jax import lax
from jax.experimental import pallas as pl
from jax.experimental.pallas import tpu as pltpu
from jax.experimental.pallas import tpu_sc as plsc

S_K = 64
W_K = 128
BATCH = 16384
NC = 2
NS = 16
LANES = 16
NW = NC * NS
ROWS_PER_W = BATCH // NW
CHUNK = 128
NCHUNK = ROWS_PER_W // CHUNK
NSLOT = 2

_mesh = plsc.VectorSubcoreMesh(
    core_axis_name="c", subcore_axis_name="s", num_cores=NC, num_subcores=NS
)

_slot_scratch = [
    pltpu.VMEM((3, CHUNK + LANES), jnp.int32),
    pltpu.VMEM((CHUNK,), jnp.int32),
    pltpu.VMEM((CHUNK, S_K), jnp.float32),
    pltpu.VMEM((CHUNK, S_K), jnp.float32),
    pltpu.VMEM((CHUNK, W_K), jnp.float32),
    pltpu.SemaphoreType.DMA,
    pltpu.SemaphoreType.DMA,
    pltpu.SemaphoreType.DMA,
]


@functools.partial(
    pl.kernel,
    out_type=jax.ShapeDtypeStruct((BATCH,), jnp.float32),
    mesh=_mesh,
    scratch_types=_slot_scratch * NSLOT + [
        pltpu.VMEM((CHUNK,), jnp.float32),
    ],
)
def _sc_forward(xT, s_tab, r_tab, w_tab, out, *scratch):
    nper = len(_slot_scratch)
    slots = [scratch[i * nper:(i + 1) * nper] for i in range(NSLOT)]
    outv = scratch[NSLOT * nper]

    wid = lax.axis_index("s") * NC + lax.axis_index("c")
    lane = lax.iota(jnp.int32, LANES)

    _dnums = lax.GatherDimensionNumbers(
        offset_dims=(), collapsed_slice_dims=(0,), start_index_map=(0,)
    )

    def _lane_shuffle(v, idx):
        return lax.gather(
            v, idx[:, None], _dnums, slice_sizes=(1,),
            mode=lax.GatherScatterMode.PROMISE_IN_BOUNDS,
        )

    def issue(c, slot):
        xbuf, idx2, srows, rrows, wrows, sem_s, sem_r, sem_w = slot
        base = wid * ROWS_PER_W + c * CHUNK
        for col in range(3):
            pltpu.sync_copy(xT.at[pl.ds(col, 1), pl.ds(base, CHUNK)],
                            xbuf.at[pl.ds(col, 1), pl.ds(0, CHUNK)])

        def widx_body(g, carry):
            idx2[pl.ds(g * LANES, LANES)] = xbuf[2, pl.ds(g * LANES, LANES)]
            return carry

        lax.fori_loop(0, CHUNK // LANES, widx_body, 0)
        pltpu.async_copy(w_tab.at[idx2], wrows, sem_w)

        def dma_body(g, carry):
            v0 = xbuf[0, pl.ds(g * LANES, LANES)]
            v1 = xbuf[1, pl.ds(g * LANES, LANES)]
            for l in range(LANES):
                j = g * LANES + l
                pltpu.async_copy(s_tab.at[pl.ds(v0[l], 1), :],
                                 srows.at[pl.ds(j, 1), :], sem_s)
                pltpu.async_copy(r_tab.at[pl.ds(v1[l], 1), :],
                                 rrows.at[pl.ds(j, 1), :], sem_r)
            return carry

        lax.fori_loop(0, CHUNK // LANES, dma_body, 0)

    def drain(slot):
        _, _, srows, rrows, wrows, sem_s, sem_r, sem_w = slot
        pltpu.make_async_copy(s_tab.at[pl.ds(0, CHUNK), :], srows,
                              sem_s).wait()
        pltpu.make_async_copy(r_tab.at[pl.ds(0, CHUNK), :], rrows,
                              sem_r).wait()
        pltpu.make_async_copy(w_tab.at[pl.ds(0, CHUNK), :], wrows,
                              sem_w).wait()

    def compute(c, slot):
        _, _, srows, rrows, wrows, _, _, _ = slot
        base = wid * ROWS_PER_W + c * CHUNK

        def group_body(g, carry):
            def row_body(j, acc_out):
                i = g * LANES + j
                acc = srows[i, pl.ds(0, LANES)] * wrows[i, pl.ds(0, LANES)]
                for k in range(1, S_K // LANES):
                    acc = acc + (srows[i, pl.ds(k * LANES, LANES)]
                                 * wrows[i, pl.ds(k * LANES, LANES)])
                for k in range(S_K // LANES):
                    acc = acc + (rrows[i, pl.ds(k * LANES, LANES)]
                                 * wrows[i, pl.ds(S_K + k * LANES, LANES)])
                for d in (8, 4, 2, 1):
                    acc = acc + _lane_shuffle(acc, lane ^ d)
                return jnp.where(lane == j, acc, acc_out)

            accs = lax.fori_loop(
                0, LANES, row_body, jnp.zeros((LANES,), jnp.float32)
            )
            outv[pl.ds(g * LANES, LANES)] = 1.0 / (1.0 + jnp.exp(-accs))
            return carry

        lax.fori_loop(0, CHUNK // LANES, group_body, 0)
        pltpu.sync_copy(outv, out.at[pl.ds(base, CHUNK)])

    issue(0, slots[0])
    for c in range(NCHUNK):
        if c + 1 < NCHUNK:
            issue(c + 1, slots[(c + 1) % NSLOT])
        drain(slots[c % NSLOT])
        compute(c, slots[c % NSLOT])


def kernel(X, s_embeds, r_embeds, w_embeds):
    xT = X.astype(jnp.int32).T
    return _sc_forward(xT, s_embeds, r_embeds, w_embeds)
